# Initial kernel scaffold; baseline (speedup 1.0000x reference)
import jax
import jax.numpy as jnp
from jax.experimental import pallas as pl


def kernel(x, w_gate, experts_hidden_w, experts_hidden_bias, experts_w, experts_bias, output_weight, norm_w, norm_bias):
    raise NotImplementedError("write your pallas kernel here")


# speedup vs baseline: 1.2410x; 1.2410x over previous
import functools

import jax
import jax.numpy as jnp
from jax.experimental import pallas as pl
from jax.experimental.pallas import tpu as pltpu

T = 2048
D = 1024
H = 1024
O = 1024
E = 8
EPS = 1e-6

BT = 256


def _moe_body(x_ref, wg_ref, ehw_ref, ehb_ref, ew_ref, eb_ref, ow_ref,
              nw_ref, nb_ref, out_ref, y_acc, gates_s):
    e = pl.program_id(0)
    i = pl.program_id(1)
    rows = pl.ds(i * BT, BT)

    xb = x_ref[...]
    mu = jnp.mean(xb, axis=1, keepdims=True)
    var = jnp.mean((xb - mu) ** 2, axis=1, keepdims=True)
    xn = (xb - mu) * jax.lax.rsqrt(var + EPS) * nw_ref[...] + nb_ref[...]

    col = jax.lax.broadcasted_iota(jnp.int32, (BT, E), 1)

    @pl.when(e == 0)
    def _():
        logits = jnp.dot(xn, wg_ref[...], preferred_element_type=jnp.float32)
        m1 = jnp.max(logits, axis=1, keepdims=True)
        i1 = jnp.argmax(logits, axis=1)[:, None]
        masked = jnp.where(col == i1, -jnp.inf, logits)
        m2 = jnp.max(masked, axis=1, keepdims=True)
        i2 = jnp.argmax(masked, axis=1)[:, None]
        g1 = 1.0 / (1.0 + jnp.exp(m2 - m1))
        g2 = 1.0 - g1
        gates = jnp.where(col == i1, g1, jnp.where(col == i2, g2, 0.0))
        gates_s[rows, :] = gates
        y_acc[rows, :] = jnp.dot(xn, ow_ref[...],
                                 preferred_element_type=jnp.float32)

    wh = ehw_ref[0]
    h = jax.lax.dot_general(xn, wh, (((1,), (1,)), ((), ())),
                            preferred_element_type=jnp.float32) + ehb_ref[0]
    act = h * jax.nn.sigmoid(h)
    wo = ew_ref[0]
    o = jax.lax.dot_general(act, wo, (((1,), (1,)), ((), ())),
                            preferred_element_type=jnp.float32) + eb_ref[0]
    gcol = jnp.sum(gates_s[rows, :] * (col == e).astype(jnp.float32),
                   axis=1, keepdims=True)
    y_acc[rows, :] += gcol * o

    @pl.when(e == E - 1)
    def _():
        out_ref[...] = y_acc[rows, :]


@functools.partial(jax.jit, static_argnames=("interpret",))
def _moe(x, w_gate, ehw, ehb, ew, eb, ow, nw, nb, interpret=False):
    grid = (E, T // BT)
    return pl.pallas_call(
        _moe_body,
        grid=grid,
        in_specs=[
            pl.BlockSpec((BT, D), lambda e, i: (i, 0)),
            pl.BlockSpec((D, E), lambda e, i: (0, 0)),
            pl.BlockSpec((1, H, D), lambda e, i: (e, 0, 0)),
            pl.BlockSpec((1, 1, H), lambda e, i: (e, 0, 0)),
            pl.BlockSpec((1, O, H), lambda e, i: (e, 0, 0)),
            pl.BlockSpec((1, 1, O), lambda e, i: (e, 0, 0)),
            pl.BlockSpec((D, O), lambda e, i: (0, 0)),
            pl.BlockSpec((1, D), lambda e, i: (0, 0)),
            pl.BlockSpec((1, D), lambda e, i: (0, 0)),
        ],
        out_specs=pl.BlockSpec((BT, O), lambda e, i: (i, 0)),
        out_shape=jax.ShapeDtypeStruct((T, O), jnp.float32),
        scratch_shapes=[
            pltpu.VMEM((T, O), jnp.float32),
            pltpu.VMEM((T, E), jnp.float32),
        ],
        interpret=interpret,
    )(x, w_gate, ehw, ehb, ew, eb, ow, nw, nb)


def kernel(x, w_gate, experts_hidden_w, experts_hidden_bias, experts_w,
           experts_bias, output_weight, norm_w, norm_bias):
    nw = norm_w.reshape(1, D)
    nb = norm_bias.reshape(1, D)
    return _moe(x, w_gate, experts_hidden_w,
                experts_hidden_bias.reshape(E, 1, H), experts_w,
                experts_bias.reshape(E, 1, O), output_weight, nw, nb)
